# Initial kernel scaffold; baseline (speedup 1.0000x reference)
import functools

import jax
import jax.numpy as jnp
from jax import lax
from jax.experimental import pallas as pl
from jax.experimental.pallas import tpu as pltpu
from jax.experimental.pallas import tpu_sc as plsc

N = 10000
E = 320000
D = 128

NTILES = 32
R = 10240
RPT = R // 16
CHUNK = 128
EPW = 10240
NCHUNK = EPW // CHUNK
EPAD = NTILES * EPW
DUMMY = N

_mesh = plsc.VectorSubcoreMesh(core_axis_name="c", subcore_axis_name="s")
_prec = jax.lax.Precision.HIGHEST


@functools.partial(
    pl.kernel,
    out_type=jax.ShapeDtypeStruct((2, R), jnp.float32),
    mesh=_mesh,
    scratch_types=[
        pltpu.VMEM((NCHUNK, CHUNK), jnp.int32),
        pltpu.VMEM((CHUNK,), jnp.float32),
        pltpu.VMEM((RPT,), jnp.float32),
        pltpu.SemaphoreType.DMA,
        pltpu.VMEM_SHARED((R,), jnp.float32),
    ],
)
def _deg_kernel(dst_hbm, out_hbm, dst_v, ones_v, zeros_v, sem, hist_sh):
    c = lax.axis_index("c")
    s = lax.axis_index("s")
    w = c * 16 + s
    pltpu.sync_copy(dst_hbm.at[pl.ds(w * NCHUNK, NCHUNK)], dst_v)

    def _z16(k, carry):
        zeros_v[pl.ds(k * 16, 16)] = jnp.zeros((16,), jnp.float32)
        return carry

    lax.fori_loop(0, RPT // 16, _z16, 0)

    def _o16(k, carry):
        ones_v[pl.ds(k * 16, 16)] = jnp.ones((16,), jnp.float32)
        return carry

    lax.fori_loop(0, CHUNK // 16, _o16, 0)

    pltpu.sync_copy(zeros_v, hist_sh.at[pl.ds(s * RPT, RPT)])
    plsc.subcore_barrier()

    def _fire8(g, carry):
        for b in range(8):
            pltpu.make_async_copy(
                ones_v, hist_sh.at[dst_v.at[g * 8 + b]], sem
            ).start(add=True)
        for b in range(8):
            pltpu.make_async_copy(
                ones_v, hist_sh.at[dst_v.at[g * 8 + b]], sem
            ).wait()
        return carry

    lax.fori_loop(0, NCHUNK // 8, _fire8, 0)
    plsc.subcore_barrier()
    pltpu.sync_copy(
        hist_sh.at[pl.ds(s * RPT, RPT)], out_hbm.at[c, pl.ds(s * RPT, RPT)]
    )


@functools.partial(
    pl.kernel,
    out_type=jax.ShapeDtypeStruct((2, R, D), jnp.float32),
    mesh=_mesh,
    scratch_types=[
        pltpu.VMEM((NCHUNK, CHUNK), jnp.int32),
        pltpu.VMEM((NCHUNK, CHUNK), jnp.int32),
        pltpu.VMEM((2, CHUNK, D), jnp.float32),
        pltpu.SemaphoreType.DMA,
        pltpu.SemaphoreType.DMA,
        pltpu.VMEM_SHARED((R, D), jnp.float32),
    ],
)
def _agg_kernel(y_hbm, src_hbm, dst_hbm, out_hbm, src_v, dst_v, rows_v,
                sem0, sem1, acc_sh):
    c = lax.axis_index("c")
    s = lax.axis_index("s")
    w = c * 16 + s
    pltpu.sync_copy(src_hbm.at[pl.ds(w * NCHUNK, NCHUNK)], src_v)
    pltpu.sync_copy(dst_hbm.at[pl.ds(w * NCHUNK, NCHUNK)], dst_v)
    pltpu.sync_copy(y_hbm.at[pl.ds(s * RPT, RPT)], acc_sh.at[pl.ds(s * RPT, RPT)])
    plsc.subcore_barrier()

    sems = (sem0, sem1)
    pltpu.make_async_copy(y_hbm.at[src_v.at[0]], rows_v.at[0], sem0).start()

    def _body(t, carry):
        for b in range(2):
            j = t * 2 + b
            nxt = j + 1

            @pl.when(nxt < NCHUNK)
            def _():
                pltpu.make_async_copy(
                    y_hbm.at[src_v.at[nxt]], rows_v.at[1 - b], sems[1 - b]
                ).start()

            pltpu.make_async_copy(
                y_hbm.at[src_v.at[j]], rows_v.at[b], sems[b]
            ).wait()
            pltpu.sync_copy(rows_v.at[b], acc_sh.at[dst_v.at[j]], add=True)
        return carry

    lax.fori_loop(0, NCHUNK // 2, _body, 0)
    plsc.subcore_barrier()
    pltpu.sync_copy(
        acc_sh.at[pl.ds(s * RPT, RPT)], out_hbm.at[c, pl.ds(s * RPT, RPT)]
    )


BLK = 512


def _scale_mm_body(degp_ref, x_ref, w_ref, y_ref, dinv_ref):
    deg = degp_ref[0] + degp_ref[1] + 1.0
    dinv = lax.rsqrt(deg)
    xw = jnp.dot(x_ref[...], w_ref[...], preferred_element_type=jnp.float32,
                 precision=_prec)
    y_ref[...] = xw * dinv
    dinv_ref[...] = dinv


def _scale_matmul(degp, xp, W1):
    return pl.pallas_call(
        _scale_mm_body,
        grid=(R // BLK,),
        in_specs=[
            pl.BlockSpec((2, BLK, 1), lambda i: (0, i, 0)),
            pl.BlockSpec((BLK, D), lambda i: (i, 0)),
            pl.BlockSpec((D, D), lambda i: (0, 0)),
        ],
        out_specs=[
            pl.BlockSpec((BLK, D), lambda i: (i, 0)),
            pl.BlockSpec((BLK, 1), lambda i: (i, 0)),
        ],
        out_shape=[
            jax.ShapeDtypeStruct((R, D), jnp.float32),
            jax.ShapeDtypeStruct((R, 1), jnp.float32),
        ],
    )(degp, xp, W1)


def _mid_body(p_ref, y1_ref, dinv_ref, b1_ref, w2_ref, y2_ref):
    agg = p_ref[0] + p_ref[1] - y1_ref[...]
    h = jnp.maximum(agg * dinv_ref[...] + b1_ref[...], 0.0)
    y2_ref[...] = jnp.dot(h, w2_ref[...], preferred_element_type=jnp.float32,
                          precision=_prec) * dinv_ref[...]


def _mid(p, y1, dinv, b1, W2):
    return pl.pallas_call(
        _mid_body,
        grid=(R // BLK,),
        in_specs=[
            pl.BlockSpec((2, BLK, D), lambda i: (0, i, 0)),
            pl.BlockSpec((BLK, D), lambda i: (i, 0)),
            pl.BlockSpec((BLK, 1), lambda i: (i, 0)),
            pl.BlockSpec((1, D), lambda i: (0, 0)),
            pl.BlockSpec((D, D), lambda i: (0, 0)),
        ],
        out_specs=pl.BlockSpec((BLK, D), lambda i: (i, 0)),
        out_shape=jax.ShapeDtypeStruct((R, D), jnp.float32),
    )(p, y1, dinv, b1, W2)


def _final_body(q_ref, y2_ref, dinv_ref, b2_ref, out_ref):
    agg = q_ref[0] + q_ref[1] - y2_ref[...]
    out_ref[...] = agg * dinv_ref[...] + b2_ref[...]


def _final(q, y2, dinv, b2):
    return pl.pallas_call(
        _final_body,
        grid=(R // BLK,),
        in_specs=[
            pl.BlockSpec((2, BLK, D), lambda i: (0, i, 0)),
            pl.BlockSpec((BLK, D), lambda i: (i, 0)),
            pl.BlockSpec((BLK, 1), lambda i: (i, 0)),
            pl.BlockSpec((1, D), lambda i: (0, 0)),
        ],
        out_specs=pl.BlockSpec((BLK, D), lambda i: (i, 0)),
        out_shape=jax.ShapeDtypeStruct((R, D), jnp.float32),
    )(q, y2, dinv, b2)


def kernel(x, edge_index, W1, b1, W2, b2):
    pad = jnp.full((EPAD - E,), DUMMY, dtype=jnp.int32)
    src2d = jnp.reshape(jnp.concatenate([edge_index[0], pad]), (EPAD // CHUNK, CHUNK))
    dst2d = jnp.reshape(jnp.concatenate([edge_index[1], pad]), (EPAD // CHUNK, CHUNK))
    xp = jnp.pad(x, ((0, R - N), (0, 0)))

    degp = _deg_kernel(dst2d).reshape(2, R, 1)
    y1, dinv = _scale_matmul(degp, xp, W1)
    p = _agg_kernel(y1, src2d, dst2d)
    y2 = _mid(p, y1, dinv, b1.reshape(1, D), W2)
    q = _agg_kernel(y2, src2d, dst2d)
    out = _final(q, y2, dinv, b2.reshape(1, D))
    return out[:N]


# speedup vs baseline: 9.1447x; 9.1447x over previous
import functools

import jax
import jax.numpy as jnp
from jax import lax
from jax.experimental import pallas as pl
from jax.experimental.pallas import tpu as pltpu
from jax.experimental.pallas import tpu_sc as plsc

N = 10000
E = 320000
D = 128

NTILES = 32
R = 10240
RPT = R // 16
CHUNK = 80
EPW = 10240
NCHUNK = EPW // CHUNK
EPAD = NTILES * EPW
DUMMY = N

_mesh = plsc.VectorSubcoreMesh(core_axis_name="c", subcore_axis_name="s")
_prec = jax.lax.Precision.HIGHEST


@functools.partial(
    pl.kernel,
    out_type=jax.ShapeDtypeStruct((2, R), jnp.float32),
    mesh=_mesh,
    scratch_types=[
        pltpu.VMEM((NCHUNK, CHUNK), jnp.int32),
        pltpu.VMEM((CHUNK,), jnp.float32),
        pltpu.VMEM((RPT,), jnp.float32),
        pltpu.SemaphoreType.DMA,
        pltpu.VMEM_SHARED((R,), jnp.float32),
    ],
)
def _deg_kernel(dst_hbm, out_hbm, dst_v, ones_v, zeros_v, sem, hist_sh):
    c = lax.axis_index("c")
    s = lax.axis_index("s")
    w = c * 16 + s
    pltpu.sync_copy(dst_hbm.at[pl.ds(w * NCHUNK, NCHUNK)], dst_v)

    def _z16(k, carry):
        zeros_v[pl.ds(k * 16, 16)] = jnp.zeros((16,), jnp.float32)
        return carry

    lax.fori_loop(0, RPT // 16, _z16, 0)

    def _o16(k, carry):
        ones_v[pl.ds(k * 16, 16)] = jnp.ones((16,), jnp.float32)
        return carry

    lax.fori_loop(0, CHUNK // 16, _o16, 0)

    pltpu.sync_copy(zeros_v, hist_sh.at[pl.ds(s * RPT, RPT)])
    plsc.subcore_barrier()

    def _fire8(g, carry):
        for b in range(8):
            pltpu.make_async_copy(
                ones_v, hist_sh.at[dst_v.at[g * 8 + b]], sem
            ).start(add=True)
        for b in range(8):
            pltpu.make_async_copy(
                ones_v, hist_sh.at[dst_v.at[g * 8 + b]], sem
            ).wait()
        return carry

    lax.fori_loop(0, NCHUNK // 8, _fire8, 0)
    plsc.subcore_barrier()
    pltpu.sync_copy(
        hist_sh.at[pl.ds(s * RPT, RPT)], out_hbm.at[c, pl.ds(s * RPT, RPT)]
    )


@functools.partial(
    pl.kernel,
    out_type=jax.ShapeDtypeStruct((2, R, D), jnp.float32),
    mesh=_mesh,
    scratch_types=[
        pltpu.VMEM((NCHUNK // 2, CHUNK), jnp.int32),
        pltpu.VMEM((NCHUNK // 2, CHUNK), jnp.int32),
        pltpu.VMEM((2, CHUNK, D), jnp.float32),
        pltpu.SemaphoreType.DMA,
        pltpu.SemaphoreType.DMA,
        pltpu.VMEM_SHARED((R, D), jnp.float32),
    ],
)
def _agg_kernel(y_hbm, src_hbm, dst_hbm, out_hbm, src_v, dst_v, rows_v,
                sem0, sem1, acc_sh):
    c = lax.axis_index("c")
    s = lax.axis_index("s")
    w = c * 16 + s
    pltpu.sync_copy(y_hbm.at[pl.ds(s * RPT, RPT)], acc_sh.at[pl.ds(s * RPT, RPT)])
    plsc.subcore_barrier()

    sems = (sem0, sem1)
    half = NCHUNK // 2
    for h in range(2):
        base = w * NCHUNK + h * half
        pltpu.sync_copy(src_hbm.at[pl.ds(base, half)], src_v)
        pltpu.sync_copy(dst_hbm.at[pl.ds(base, half)], dst_v)
        pltpu.make_async_copy(y_hbm.at[src_v.at[0]], rows_v.at[0], sem0).start()

        def _body(t, carry):
            for b in range(2):
                j = t * 2 + b
                nxt = j + 1

                @pl.when(nxt < half)
                def _():
                    pltpu.make_async_copy(
                        y_hbm.at[src_v.at[nxt]], rows_v.at[1 - b], sems[1 - b]
                    ).start()

                pltpu.make_async_copy(
                    y_hbm.at[src_v.at[j]], rows_v.at[b], sems[b]
                ).wait()
                pltpu.sync_copy(rows_v.at[b], acc_sh.at[dst_v.at[j]], add=True)
            return carry

        lax.fori_loop(0, half // 2, _body, 0)
    plsc.subcore_barrier()
    pltpu.sync_copy(
        acc_sh.at[pl.ds(s * RPT, RPT)], out_hbm.at[c, pl.ds(s * RPT, RPT)]
    )


BLK = 512


def _scale_mm_body(degp_ref, x_ref, w_ref, y_ref, dinv_ref):
    deg = degp_ref[0] + degp_ref[1] + 1.0
    dinv = lax.rsqrt(deg)
    xw = jnp.dot(x_ref[...], w_ref[...], preferred_element_type=jnp.float32,
                 precision=_prec)
    y_ref[...] = xw * dinv
    dinv_ref[...] = dinv


def _scale_matmul(degp, xp, W1):
    return pl.pallas_call(
        _scale_mm_body,
        grid=(R // BLK,),
        in_specs=[
            pl.BlockSpec((2, BLK, 1), lambda i: (0, i, 0)),
            pl.BlockSpec((BLK, D), lambda i: (i, 0)),
            pl.BlockSpec((D, D), lambda i: (0, 0)),
        ],
        out_specs=[
            pl.BlockSpec((BLK, D), lambda i: (i, 0)),
            pl.BlockSpec((BLK, 1), lambda i: (i, 0)),
        ],
        out_shape=[
            jax.ShapeDtypeStruct((R, D), jnp.float32),
            jax.ShapeDtypeStruct((R, 1), jnp.float32),
        ],
    )(degp, xp, W1)


def _mid_body(p_ref, y1_ref, dinv_ref, b1_ref, w2_ref, y2_ref):
    agg = p_ref[0] + p_ref[1] - y1_ref[...]
    h = jnp.maximum(agg * dinv_ref[...] + b1_ref[...], 0.0)
    y2_ref[...] = jnp.dot(h, w2_ref[...], preferred_element_type=jnp.float32,
                          precision=_prec) * dinv_ref[...]


def _mid(p, y1, dinv, b1, W2):
    return pl.pallas_call(
        _mid_body,
        grid=(R // BLK,),
        in_specs=[
            pl.BlockSpec((2, BLK, D), lambda i: (0, i, 0)),
            pl.BlockSpec((BLK, D), lambda i: (i, 0)),
            pl.BlockSpec((BLK, 1), lambda i: (i, 0)),
            pl.BlockSpec((1, D), lambda i: (0, 0)),
            pl.BlockSpec((D, D), lambda i: (0, 0)),
        ],
        out_specs=pl.BlockSpec((BLK, D), lambda i: (i, 0)),
        out_shape=jax.ShapeDtypeStruct((R, D), jnp.float32),
    )(p, y1, dinv, b1, W2)


def _final_body(q_ref, y2_ref, dinv_ref, b2_ref, out_ref):
    agg = q_ref[0] + q_ref[1] - y2_ref[...]
    out_ref[...] = agg * dinv_ref[...] + b2_ref[...]


def _final(q, y2, dinv, b2):
    return pl.pallas_call(
        _final_body,
        grid=(R // BLK,),
        in_specs=[
            pl.BlockSpec((2, BLK, D), lambda i: (0, i, 0)),
            pl.BlockSpec((BLK, D), lambda i: (i, 0)),
            pl.BlockSpec((BLK, 1), lambda i: (i, 0)),
            pl.BlockSpec((1, D), lambda i: (0, 0)),
        ],
        out_specs=pl.BlockSpec((BLK, D), lambda i: (i, 0)),
        out_shape=jax.ShapeDtypeStruct((R, D), jnp.float32),
    )(q, y2, dinv, b2)


def kernel(x, edge_index, W1, b1, W2, b2):
    pad = jnp.full((EPAD - E,), DUMMY, dtype=jnp.int32)
    src2d = jnp.reshape(jnp.concatenate([edge_index[0], pad]), (EPAD // CHUNK, CHUNK))
    dst2d = jnp.reshape(jnp.concatenate([edge_index[1], pad]), (EPAD // CHUNK, CHUNK))
    xp = jnp.pad(x, ((0, R - N), (0, 0)))

    degp = _deg_kernel(dst2d).reshape(2, R, 1)
    y1, dinv = _scale_matmul(degp, xp, W1)
    p = _agg_kernel(y1, src2d, dst2d)
    y2 = _mid(p, y1, dinv, b1.reshape(1, D), W2)
    q = _agg_kernel(y2, src2d, dst2d)
    out = _final(q, y2, dinv, b2.reshape(1, D))
    return out[:N]


# speedup vs baseline: 27.8625x; 3.0469x over previous
import functools

import jax
import jax.numpy as jnp
from jax import lax
from jax.experimental import pallas as pl
from jax.experimental.pallas import tpu as pltpu
from jax.experimental.pallas import tpu_sc as plsc

N = 10000
E = 320000
D = 128

NTILES = 32
R = 10240
RPT = R // 16
CHUNK = 80
EPW = 10240
NCHUNK = EPW // CHUNK
EPAD = NTILES * EPW
DUMMY = N

_mesh = plsc.VectorSubcoreMesh(core_axis_name="c", subcore_axis_name="s")
_prec = jax.lax.Precision.HIGHEST


@functools.partial(
    pl.kernel,
    out_type=jax.ShapeDtypeStruct((2, R), jnp.float32),
    mesh=_mesh,
    scratch_types=[
        pltpu.VMEM((NCHUNK, CHUNK), jnp.int32),
        pltpu.VMEM((CHUNK,), jnp.float32),
        pltpu.VMEM((RPT,), jnp.float32),
        pltpu.SemaphoreType.DMA,
        pltpu.VMEM_SHARED((R,), jnp.float32),
    ],
)
def _deg_kernel(dst_hbm, out_hbm, dst_v, ones_v, zeros_v, sem, hist_sh):
    c = lax.axis_index("c")
    s = lax.axis_index("s")
    w = c * 16 + s
    pltpu.sync_copy(dst_hbm.at[pl.ds(w * NCHUNK, NCHUNK)], dst_v)

    def _z16(k, carry):
        zeros_v[pl.ds(k * 16, 16)] = jnp.zeros((16,), jnp.float32)
        return carry

    lax.fori_loop(0, RPT // 16, _z16, 0)

    def _o16(k, carry):
        ones_v[pl.ds(k * 16, 16)] = jnp.ones((16,), jnp.float32)
        return carry

    lax.fori_loop(0, CHUNK // 16, _o16, 0)

    pltpu.sync_copy(zeros_v, hist_sh.at[pl.ds(s * RPT, RPT)])
    plsc.subcore_barrier()

    def _fire8(g, carry):
        for b in range(8):
            pltpu.make_async_copy(
                ones_v, hist_sh.at[dst_v.at[g * 8 + b]], sem
            ).start(add=True)
        for b in range(8):
            pltpu.make_async_copy(
                ones_v, hist_sh.at[dst_v.at[g * 8 + b]], sem
            ).wait()
        return carry

    lax.fori_loop(0, NCHUNK // 8, _fire8, 0)
    plsc.subcore_barrier()
    pltpu.sync_copy(
        hist_sh.at[pl.ds(s * RPT, RPT)], out_hbm.at[c, pl.ds(s * RPT, RPT)]
    )


@functools.partial(
    pl.kernel,
    out_type=jax.ShapeDtypeStruct((2, R, D), jnp.float32),
    mesh=_mesh,
    scratch_types=[
        pltpu.VMEM((NCHUNK // 2, CHUNK), jnp.int32),
        pltpu.VMEM((NCHUNK // 2, CHUNK), jnp.int32),
        pltpu.VMEM((2, CHUNK, D), jnp.float32),
        pltpu.SemaphoreType.DMA,
        pltpu.SemaphoreType.DMA,
        pltpu.VMEM_SHARED((R, D), jnp.float32),
    ],
)
def _agg_kernel(y_hbm, src_hbm, dst_hbm, out_hbm, src_v, dst_v, rows_v,
                sem0, sem1, acc_sh):
    c = lax.axis_index("c")
    s = lax.axis_index("s")
    w = c * 16 + s
    pltpu.sync_copy(y_hbm.at[pl.ds(s * RPT, RPT)], acc_sh.at[pl.ds(s * RPT, RPT)])
    plsc.subcore_barrier()

    sems = (sem0, sem1)
    half = NCHUNK // 2
    for h in range(2):
        base = w * NCHUNK + h * half
        pltpu.sync_copy(src_hbm.at[pl.ds(base, half)], src_v)
        pltpu.sync_copy(dst_hbm.at[pl.ds(base, half)], dst_v)
        pltpu.make_async_copy(y_hbm.at[src_v.at[0]], rows_v.at[0], sem0).start()

        def _body(t, carry):
            for b in range(2):
                j = t * 2 + b
                nxt = j + 1

                @pl.when(nxt < half)
                def _():
                    pltpu.make_async_copy(
                        y_hbm.at[src_v.at[nxt]], rows_v.at[1 - b], sems[1 - b]
                    ).start()

                pltpu.make_async_copy(
                    y_hbm.at[src_v.at[j]], rows_v.at[b], sems[b]
                ).wait()
                pltpu.sync_copy(rows_v.at[b], acc_sh.at[dst_v.at[j]], add=True)
            return carry

        lax.fori_loop(0, half // 2, _body, 0)
    plsc.subcore_barrier()
    pltpu.sync_copy(
        acc_sh.at[pl.ds(s * RPT, RPT)], out_hbm.at[c, pl.ds(s * RPT, RPT)]
    )


BLK = 512


def _scale_mm_body(degp_ref, x_ref, w_ref, y_ref, dinv_ref):
    deg = degp_ref[0] + degp_ref[1] + 1.0
    dinv = lax.rsqrt(deg)
    xw = jnp.dot(x_ref[...], w_ref[...], preferred_element_type=jnp.float32,
                 precision=_prec)
    y_ref[...] = xw * dinv
    dinv_ref[...] = dinv


def _scale_matmul(degp, xp, W1):
    return pl.pallas_call(
        _scale_mm_body,
        grid=(R // BLK,),
        in_specs=[
            pl.BlockSpec((2, BLK, 1), lambda i: (0, i, 0)),
            pl.BlockSpec((BLK, D), lambda i: (i, 0)),
            pl.BlockSpec((D, D), lambda i: (0, 0)),
        ],
        out_specs=[
            pl.BlockSpec((BLK, D), lambda i: (i, 0)),
            pl.BlockSpec((BLK, 1), lambda i: (i, 0)),
        ],
        out_shape=[
            jax.ShapeDtypeStruct((R, D), jnp.float32),
            jax.ShapeDtypeStruct((R, 1), jnp.float32),
        ],
    )(degp, xp, W1)


def _mid_body(p_ref, y1_ref, dinv_ref, b1_ref, w2_ref, y2_ref):
    agg = p_ref[0] + p_ref[1] - y1_ref[...]
    h = jnp.maximum(agg * dinv_ref[...] + b1_ref[...], 0.0)
    y2_ref[...] = jnp.dot(h, w2_ref[...], preferred_element_type=jnp.float32,
                          precision=_prec) * dinv_ref[...]


def _mid(p, y1, dinv, b1, W2):
    return pl.pallas_call(
        _mid_body,
        grid=(R // BLK,),
        in_specs=[
            pl.BlockSpec((2, BLK, D), lambda i: (0, i, 0)),
            pl.BlockSpec((BLK, D), lambda i: (i, 0)),
            pl.BlockSpec((BLK, 1), lambda i: (i, 0)),
            pl.BlockSpec((1, D), lambda i: (0, 0)),
            pl.BlockSpec((D, D), lambda i: (0, 0)),
        ],
        out_specs=pl.BlockSpec((BLK, D), lambda i: (i, 0)),
        out_shape=jax.ShapeDtypeStruct((R, D), jnp.float32),
    )(p, y1, dinv, b1, W2)


def _final_body(q_ref, y2_ref, dinv_ref, b2_ref, out_ref):
    agg = q_ref[0] + q_ref[1] - y2_ref[...]
    out_ref[...] = agg * dinv_ref[...] + b2_ref[...]


def _final(q, y2, dinv, b2):
    return pl.pallas_call(
        _final_body,
        grid=(R // BLK,),
        in_specs=[
            pl.BlockSpec((2, BLK, D), lambda i: (0, i, 0)),
            pl.BlockSpec((BLK, D), lambda i: (i, 0)),
            pl.BlockSpec((BLK, 1), lambda i: (i, 0)),
            pl.BlockSpec((1, D), lambda i: (0, 0)),
        ],
        out_specs=pl.BlockSpec((BLK, D), lambda i: (i, 0)),
        out_shape=jax.ShapeDtypeStruct((R, D), jnp.float32),
    )(q, y2, dinv, b2)


def kernel(x, edge_index, W1, b1, W2, b2):
    pad = DUMMY + jnp.arange(EPAD - E, dtype=jnp.int32) % (R - N)
    src2d = jnp.reshape(jnp.concatenate([edge_index[0], pad]), (EPAD // CHUNK, CHUNK))
    dst2d = jnp.reshape(jnp.concatenate([edge_index[1], pad]), (EPAD // CHUNK, CHUNK))
    xp = jnp.pad(x, ((0, R - N), (0, 0)))

    degp = _deg_kernel(dst2d).reshape(2, R, 1)
    y1, dinv = _scale_matmul(degp, xp, W1)
    p = _agg_kernel(y1, src2d, dst2d)
    y2 = _mid(p, y1, dinv, b1.reshape(1, D), W2)
    q = _agg_kernel(y2, src2d, dst2d)
    out = _final(q, y2, dinv, b2.reshape(1, D))
    return out[:N]


# speedup vs baseline: 29.8089x; 1.0699x over previous
import functools

import jax
import jax.numpy as jnp
from jax import lax
from jax.experimental import pallas as pl
from jax.experimental.pallas import tpu as pltpu
from jax.experimental.pallas import tpu_sc as plsc

N = 10000
E = 320000
D = 128

NTILES = 32
R = 10240
RPT = R // 16
CHUNK = 128
EPW = 10240
NCHUNK = EPW // CHUNK
EPAD = NTILES * EPW
DUMMY = N

_mesh = plsc.VectorSubcoreMesh(core_axis_name="c", subcore_axis_name="s")
_prec = jax.lax.Precision.HIGHEST


@functools.partial(
    pl.kernel,
    out_type=jax.ShapeDtypeStruct((2, R), jnp.float32),
    mesh=_mesh,
    scratch_types=[
        pltpu.VMEM((NCHUNK // 5, CHUNK), jnp.int32),
        pltpu.VMEM((CHUNK,), jnp.float32),
        pltpu.VMEM((RPT,), jnp.float32),
        pltpu.SemaphoreType.DMA,
        pltpu.VMEM_SHARED((R,), jnp.float32),
    ],
)
def _deg_kernel(dst_hbm, out_hbm, dst_v, ones_v, zeros_v, sem, hist_sh):
    c = lax.axis_index("c")
    s = lax.axis_index("s")
    w = c * 16 + s

    def _z16(k, carry):
        zeros_v[pl.ds(k * 16, 16)] = jnp.zeros((16,), jnp.float32)
        return carry

    lax.fori_loop(0, RPT // 16, _z16, 0)

    def _o16(k, carry):
        ones_v[pl.ds(k * 16, 16)] = jnp.ones((16,), jnp.float32)
        return carry

    lax.fori_loop(0, CHUNK // 16, _o16, 0)

    pltpu.sync_copy(zeros_v, hist_sh.at[pl.ds(s * RPT, RPT)])
    plsc.subcore_barrier()

    fifth = NCHUNK // 5
    for q in range(5):
        pltpu.sync_copy(dst_hbm.at[pl.ds(w * NCHUNK + q * fifth, fifth)], dst_v)

        def _fire8(g, carry):
            for b in range(8):
                pltpu.make_async_copy(
                    ones_v, hist_sh.at[dst_v.at[g * 8 + b]], sem
                ).start(add=True)
            for b in range(8):
                pltpu.make_async_copy(
                    ones_v, hist_sh.at[dst_v.at[g * 8 + b]], sem
                ).wait()
            return carry

        lax.fori_loop(0, fifth // 8, _fire8, 0)
    plsc.subcore_barrier()
    pltpu.sync_copy(
        hist_sh.at[pl.ds(s * RPT, RPT)], out_hbm.at[c, pl.ds(s * RPT, RPT)]
    )


@functools.partial(
    pl.kernel,
    out_type=jax.ShapeDtypeStruct((2, R, D), jnp.float32),
    mesh=_mesh,
    scratch_types=[
        pltpu.VMEM((NCHUNK // 2, CHUNK), jnp.int32),
        pltpu.VMEM((NCHUNK // 2, CHUNK), jnp.int32),
        pltpu.VMEM((2, CHUNK, D), jnp.float32),
        pltpu.SemaphoreType.DMA,
        pltpu.SemaphoreType.DMA,
        pltpu.SemaphoreType.DMA,
        pltpu.SemaphoreType.DMA,
        pltpu.VMEM_SHARED((R, D), jnp.float32),
    ],
)
def _agg_kernel(y_hbm, src_hbm, dst_hbm, out_hbm, src_v, dst_v, rows_v,
                gsem0, gsem1, ssem0, ssem1, acc_sh):
    c = lax.axis_index("c")
    s = lax.axis_index("s")
    w = c * 16 + s
    pltpu.sync_copy(y_hbm.at[pl.ds(s * RPT, RPT)], acc_sh.at[pl.ds(s * RPT, RPT)])
    plsc.subcore_barrier()

    gsems = (gsem0, gsem1)
    ssems = (ssem0, ssem1)
    half = NCHUNK // 2
    for h in range(2):
        base = w * NCHUNK + h * half
        pltpu.sync_copy(src_hbm.at[pl.ds(base, half)], src_v)
        pltpu.sync_copy(dst_hbm.at[pl.ds(base, half)], dst_v)
        pltpu.make_async_copy(y_hbm.at[src_v.at[0]], rows_v.at[0], gsem0).start()

        def _body(t, carry):
            for b in range(2):
                j = t * 2 + b
                nxt = j + 1

                @pl.when(j >= 1)
                def _():
                    pltpu.make_async_copy(
                        rows_v.at[1 - b], acc_sh.at[dst_v.at[0]], ssems[1 - b]
                    ).wait()

                @pl.when(nxt < half)
                def _():
                    pltpu.make_async_copy(
                        y_hbm.at[src_v.at[nxt]], rows_v.at[1 - b], gsems[1 - b]
                    ).start()

                pltpu.make_async_copy(
                    y_hbm.at[src_v.at[j]], rows_v.at[b], gsems[b]
                ).wait()
                pltpu.make_async_copy(
                    rows_v.at[b], acc_sh.at[dst_v.at[j]], ssems[b]
                ).start(add=True)
            return carry

        lax.fori_loop(0, half // 2, _body, 0)
        pltpu.make_async_copy(rows_v.at[1], acc_sh.at[dst_v.at[0]], ssem1).wait()
    plsc.subcore_barrier()
    pltpu.sync_copy(
        acc_sh.at[pl.ds(s * RPT, RPT)], out_hbm.at[c, pl.ds(s * RPT, RPT)]
    )


BLK = 512


def _scale_mm_body(degp_ref, x_ref, w_ref, y_ref, dinv_ref):
    deg = degp_ref[0] + degp_ref[1] + 1.0
    dinv = lax.rsqrt(deg)
    xw = jnp.dot(x_ref[...], w_ref[...], preferred_element_type=jnp.float32,
                 precision=_prec)
    y_ref[...] = xw * dinv
    dinv_ref[...] = dinv


def _scale_matmul(degp, xp, W1):
    return pl.pallas_call(
        _scale_mm_body,
        grid=(R // BLK,),
        in_specs=[
            pl.BlockSpec((2, BLK, 1), lambda i: (0, i, 0)),
            pl.BlockSpec((BLK, D), lambda i: (i, 0)),
            pl.BlockSpec((D, D), lambda i: (0, 0)),
        ],
        out_specs=[
            pl.BlockSpec((BLK, D), lambda i: (i, 0)),
            pl.BlockSpec((BLK, 1), lambda i: (i, 0)),
        ],
        out_shape=[
            jax.ShapeDtypeStruct((R, D), jnp.float32),
            jax.ShapeDtypeStruct((R, 1), jnp.float32),
        ],
    )(degp, xp, W1)


def _mid_body(p_ref, y1_ref, dinv_ref, b1_ref, w2_ref, y2_ref):
    agg = p_ref[0] + p_ref[1] - y1_ref[...]
    h = jnp.maximum(agg * dinv_ref[...] + b1_ref[...], 0.0)
    y2_ref[...] = jnp.dot(h, w2_ref[...], preferred_element_type=jnp.float32,
                          precision=_prec) * dinv_ref[...]


def _mid(p, y1, dinv, b1, W2):
    return pl.pallas_call(
        _mid_body,
        grid=(R // BLK,),
        in_specs=[
            pl.BlockSpec((2, BLK, D), lambda i: (0, i, 0)),
            pl.BlockSpec((BLK, D), lambda i: (i, 0)),
            pl.BlockSpec((BLK, 1), lambda i: (i, 0)),
            pl.BlockSpec((1, D), lambda i: (0, 0)),
            pl.BlockSpec((D, D), lambda i: (0, 0)),
        ],
        out_specs=pl.BlockSpec((BLK, D), lambda i: (i, 0)),
        out_shape=jax.ShapeDtypeStruct((R, D), jnp.float32),
    )(p, y1, dinv, b1, W2)


def _final_body(q_ref, y2_ref, dinv_ref, b2_ref, out_ref):
    agg = q_ref[0] + q_ref[1] - y2_ref[...]
    out_ref[...] = agg * dinv_ref[...] + b2_ref[...]


def _final(q, y2, dinv, b2):
    return pl.pallas_call(
        _final_body,
        grid=(R // BLK,),
        in_specs=[
            pl.BlockSpec((2, BLK, D), lambda i: (0, i, 0)),
            pl.BlockSpec((BLK, D), lambda i: (i, 0)),
            pl.BlockSpec((BLK, 1), lambda i: (i, 0)),
            pl.BlockSpec((1, D), lambda i: (0, 0)),
        ],
        out_specs=pl.BlockSpec((BLK, D), lambda i: (i, 0)),
        out_shape=jax.ShapeDtypeStruct((R, D), jnp.float32),
    )(q, y2, dinv, b2)


def kernel(x, edge_index, W1, b1, W2, b2):
    pad = DUMMY + jnp.arange(EPAD - E, dtype=jnp.int32) % (R - N)
    src2d = jnp.reshape(jnp.concatenate([edge_index[0], pad]), (EPAD // CHUNK, CHUNK))
    dst2d = jnp.reshape(jnp.concatenate([edge_index[1], pad]), (EPAD // CHUNK, CHUNK))
    xp = jnp.pad(x, ((0, R - N), (0, 0)))

    degp = _deg_kernel(dst2d).reshape(2, R, 1)
    y1, dinv = _scale_matmul(degp, xp, W1)
    p = _agg_kernel(y1, src2d, dst2d)
    y2 = _mid(p, y1, dinv, b1.reshape(1, D), W2)
    q = _agg_kernel(y2, src2d, dst2d)
    out = _final(q, y2, dinv, b2.reshape(1, D))
    return out[:N]
